# Initial kernel scaffold; baseline (speedup 1.0000x reference)
import jax
import jax.numpy as jnp
from jax.experimental import pallas as pl


def kernel(x, gw, gb, ew, eb):
    raise NotImplementedError("write your pallas kernel here")


# speedup vs baseline: 2.4446x; 2.4446x over previous
import functools
import jax
import jax.numpy as jnp
from jax.experimental import pallas as pl
from jax.experimental.pallas import tpu as pltpu

B = 4096
L = 768
E = 8
P = 48
PD = 16
F = 32

BLK = 512
GRID = B // BLK


def _body(x_ref, gwk_ref, gb_ref, ew3_ref, eb_ref, noise_ref,
          out_ref, disp_ref, loss_ref, hsum_ref, msum_ref):
    pid = pl.program_id(0)

    x = x_ref[...]
    r = jax.lax.broadcasted_iota(jnp.int32, (L, PD), 0)
    c = jax.lax.broadcasted_iota(jnp.int32, (L, PD), 1)
    sel = (r % PD == c).astype(jnp.float32)
    v = jnp.dot(x, sel, preferred_element_type=jnp.float32)

    gwk = gwk_ref[...]
    gb = gb_ref[...]
    noise = noise_ref[...]
    h = jnp.dot(v, gwk.T, preferred_element_type=jnp.float32)
    h = h + gb * float(P) + noise

    pi_val = jnp.max(h, axis=1)
    e_idx = jnp.argmax(h, axis=1)
    e_iota = jax.lax.broadcasted_iota(jnp.int32, (BLK, E), 1)
    mask = (e_iota == e_idx[:, None]).astype(jnp.float32)
    disp_ref[...] = mask

    ew3 = ew3_ref[...]
    cc = jax.lax.broadcasted_iota(jnp.int32, (E, 2 * F, PD), 1)
    A0 = jnp.sum(ew3 * (cc < F), axis=1)
    A1 = jnp.sum(ew3 * (cc >= F), axis=1)
    eb = eb_ref[...]
    cb = jax.lax.broadcasted_iota(jnp.int32, (E, 2 * F), 1)
    S0 = jnp.sum(eb * (cb < F), axis=1) * float(P)
    S1 = jnp.sum(eb * (cb >= F), axis=1) * float(P)

    t0 = jnp.dot(v, A0.T, preferred_element_type=jnp.float32) + S0[None, :]
    t1 = jnp.dot(v, A1.T, preferred_element_type=jnp.float32) + S1[None, :]
    o0 = pi_val * jnp.sum(t0 * mask, axis=1)
    o1 = pi_val * jnp.sum(t1 * mask, axis=1)
    out_ref[...] = jnp.concatenate([o0[:, None], o1[:, None]], axis=1)

    @pl.when(pid == 0)
    def _init():
        hsum_ref[...] = jnp.zeros_like(hsum_ref)
        msum_ref[...] = jnp.zeros_like(msum_ref)

    hsum_ref[...] += jnp.sum(h, axis=0, keepdims=True)
    msum_ref[...] += jnp.sum(mask, axis=0, keepdims=True)

    loss_ref[...] = (float(E) / float(B * B)) * jnp.sum(
        hsum_ref[...] * msum_ref[...]).reshape(1, 1)


@functools.partial(jax.jit, static_argnames=())
def _run(x2, gwk, gb2, ew3, eb, noise):
    out, disp, loss, _, _ = pl.pallas_call(
        _body,
        grid=(GRID,),
        in_specs=[
            pl.BlockSpec((BLK, L), lambda i: (i, 0)),
            pl.BlockSpec((E, PD), lambda i: (0, 0)),
            pl.BlockSpec((1, E), lambda i: (0, 0)),
            pl.BlockSpec((E, 2 * F, PD), lambda i: (0, 0, 0)),
            pl.BlockSpec((E, 2 * F), lambda i: (0, 0)),
            pl.BlockSpec((BLK, 1), lambda i: (i, 0)),
        ],
        out_specs=[
            pl.BlockSpec((BLK, 2), lambda i: (i, 0)),
            pl.BlockSpec((BLK, E), lambda i: (i, 0)),
            pl.BlockSpec((1, 1), lambda i: (0, 0)),
            pl.BlockSpec((1, E), lambda i: (0, 0)),
            pl.BlockSpec((1, E), lambda i: (0, 0)),
        ],
        out_shape=[
            jax.ShapeDtypeStruct((B, 2), jnp.float32),
            jax.ShapeDtypeStruct((B, E), jnp.float32),
            jax.ShapeDtypeStruct((1, 1), jnp.float32),
            jax.ShapeDtypeStruct((1, E), jnp.float32),
            jax.ShapeDtypeStruct((1, E), jnp.float32),
        ],
    )(x2, gwk, gb2, ew3, eb, noise)
    return out, disp, loss[0, 0]


def kernel(x, gw, gb, ew, eb):
    b = x.shape[0]
    x2 = x.reshape(b, L)
    noise = jax.random.uniform(jax.random.key(42), (b, 1), dtype=jnp.float32)
    return _run(x2, gw[:, 0, :], gb[None, :], ew[:, :, 0, :], eb, noise)
